# baseline (speedup 1.0000x reference)
import jax
import jax.numpy as jnp
from jax import lax
from jax.experimental import pallas as pl
from jax.experimental.pallas import tpu as pltpu
from jax.experimental.pallas import tpu_sc as plsc

VOCAB = 100000
N_EMBD = 1024
BLOCK = 2048
BATCH = 4
SEQ = 2048

NC = 2
NS = 16
NW = NC * NS
LANES = 16
S_PER_W = SEQ // NW
P = 8
NSTEP = S_PER_W // P
ROWS = BATCH * P
VPR = N_EMBD // LANES


def _emb_body(x_hbm, wte_hbm, wpe_hbm, out_hbm,
              ridx_v, oidx_v, tok, pos, gsem, psem, ssem, isem):
    wid = lax.axis_index("s") * NC + lax.axis_index("c")
    sbase = wid * S_PER_W

    pdesc = {}
    for t in range(2):
        pdesc[t] = pltpu.async_copy(
            wpe_hbm.at[pl.ds(sbase + t * P, P)], pos[t], psem[t])

    idesc = [pltpu.async_copy(x_hbm.at[b, pl.ds(sbase + t * P, P)],
                              ridx_v.at[pl.ds(t * ROWS + b * P, P)], isem)
             for t in range(NSTEP) for b in range(BATCH)]

    for g in range(ROWS * NSTEP // LANES):
        j = g * LANES + lax.iota(jnp.int32, LANES)
        r = lax.rem(j, P)
        b = lax.rem(lax.div(j, P), BATCH)
        t = lax.div(j, ROWS)
        orow = b * SEQ + sbase + t * P + r
        oidx_v[(g * LANES) // ROWS, pl.ds((g * LANES) % ROWS, LANES)] = orow

    for d in idesc:
        d.wait()

    def issue_gather(t):
        return pltpu.async_copy(
            wte_hbm.at[ridx_v.at[pl.ds(t * ROWS, ROWS)]],
            tok[t % 3], gsem[t % 3])

    gdesc = {0: issue_gather(0), 1: issue_gather(1)}
    sdesc = {}
    for t in range(NSTEP):
        gdesc[t].wait()
        pdesc[t].wait()

        tb, pb = tok[t % 3], pos[t % 2]

        @pl.loop(0, P)
        def _row(r):
            @plsc.parallel_loop(0, VPR, unroll=8)
            def _vec(k):
                sl = pl.ds(k * LANES, LANES)
                pv = pb[r, sl]
                for b in range(BATCH):
                    tb[b * P + r, sl] = tb[b * P + r, sl] + pv

        sdesc[t] = pltpu.async_copy(
            tb, out_hbm.at[oidx_v.at[t]], ssem[t % 3])

        if t + 2 < NSTEP:
            pdesc[t + 2] = pltpu.async_copy(
                wpe_hbm.at[pl.ds(sbase + (t + 2) * P, P)],
                pos[t % 2], psem[t % 2])

        if t == 0:
            gdesc[2] = issue_gather(2)
        elif t + 2 < NSTEP:
            sdesc[t - 1].wait()
            gdesc[t + 2] = issue_gather(t + 2)

    for t in (NSTEP - 3, NSTEP - 2, NSTEP - 1):
        sdesc[t].wait()


@jax.jit
def kernel(x, wte, wpe):
    mesh = plsc.VectorSubcoreMesh(core_axis_name="c", subcore_axis_name="s")
    run = pl.kernel(
        _emb_body,
        out_type=jax.ShapeDtypeStruct((BATCH * SEQ, N_EMBD), jnp.float32),
        mesh=mesh,
        scratch_types=[
            pltpu.VMEM((NSTEP * ROWS,), jnp.int32),
            pltpu.VMEM((NSTEP, ROWS), jnp.int32),
            [pltpu.VMEM((ROWS, N_EMBD), jnp.float32) for _ in range(3)],
            [pltpu.VMEM((P, N_EMBD), jnp.float32) for _ in range(2)],
            [pltpu.SemaphoreType.DMA for _ in range(3)],
            [pltpu.SemaphoreType.DMA for _ in range(2)],
            [pltpu.SemaphoreType.DMA for _ in range(3)],
            pltpu.SemaphoreType.DMA,
        ],
    )
    out = run(x.astype(jnp.int32), wte, wpe)
    return out.reshape(BATCH, SEQ, N_EMBD)


# speedup vs baseline: 1.0077x; 1.0077x over previous
import jax
import jax.numpy as jnp
from jax import lax
from jax.experimental import pallas as pl
from jax.experimental.pallas import tpu as pltpu
from jax.experimental.pallas import tpu_sc as plsc

VOCAB = 100000
N_EMBD = 1024
BLOCK = 2048
BATCH = 4
SEQ = 2048

NC = 2
NS = 16
NW = NC * NS
LANES = 16
S_PER_W = SEQ // NW
P = 8
NSTEP = S_PER_W // P
ROWS = BATCH * P
VPR = N_EMBD // LANES


def _emb_body(x_hbm, wte_hbm, wpe_hbm, out_hbm,
              idx_v, tok, pos, gsem, psem, ssem, isem):
    wid = lax.axis_index("s") * NC + lax.axis_index("c")
    sbase = wid * S_PER_W

    pdesc = {}
    for t in range(2):
        pdesc[t] = pltpu.async_copy(
            wpe_hbm.at[pl.ds(sbase + t * P, P)], pos[t], psem[t])

    idesc = [pltpu.async_copy(x_hbm.at[b, pl.ds(sbase, S_PER_W)],
                              idx_v.at[pl.ds(b * S_PER_W, S_PER_W)], isem)
             for b in range(BATCH)]
    for d in idesc:
        d.wait()

    def issue_gathers(t):
        tb = tok[t % 3]
        return [pltpu.async_copy(
            wte_hbm.at[idx_v.at[pl.ds(b * S_PER_W + t * P, P)]],
            tb.at[pl.ds(b * P, P)], gsem[t % 3]) for b in range(BATCH)]

    gdesc = {0: issue_gathers(0), 1: issue_gathers(1)}
    sdesc = {}
    for t in range(NSTEP):
        for d in gdesc[t]:
            d.wait()
        pdesc[t].wait()

        tb, pb = tok[t % 3], pos[t % 2]

        @plsc.parallel_loop(0, P)
        def _row(r):
            @plsc.parallel_loop(0, VPR, unroll=8)
            def _vec(k):
                sl = pl.ds(k * LANES, LANES)
                pv = pb[r, sl]
                for b in range(BATCH):
                    tb[b * P + r, sl] = tb[b * P + r, sl] + pv

        sdesc[t] = [pltpu.async_copy(
            tb.at[pl.ds(b * P, P)],
            out_hbm.at[pl.ds(b * SEQ + sbase + t * P, P)],
            ssem[t % 3]) for b in range(BATCH)]

        if t + 2 < NSTEP:
            pdesc[t + 2] = pltpu.async_copy(
                wpe_hbm.at[pl.ds(sbase + (t + 2) * P, P)],
                pos[t % 2], psem[t % 2])

        if t == 0:
            gdesc[2] = issue_gathers(2)
        elif t + 2 < NSTEP:
            for d in sdesc[t - 1]:
                d.wait()
            gdesc[t + 2] = issue_gathers(t + 2)

    for t in (NSTEP - 3, NSTEP - 2, NSTEP - 1):
        for d in sdesc[t]:
            d.wait()


@jax.jit
def kernel(x, wte, wpe):
    mesh = plsc.VectorSubcoreMesh(core_axis_name="c", subcore_axis_name="s")
    run = pl.kernel(
        _emb_body,
        out_type=jax.ShapeDtypeStruct((BATCH * SEQ, N_EMBD), jnp.float32),
        mesh=mesh,
        scratch_types=[
            pltpu.VMEM((BATCH * S_PER_W,), jnp.int32),
            [pltpu.VMEM((ROWS, N_EMBD), jnp.float32) for _ in range(3)],
            [pltpu.VMEM((P, N_EMBD), jnp.float32) for _ in range(2)],
            [pltpu.SemaphoreType.DMA for _ in range(3)],
            [pltpu.SemaphoreType.DMA for _ in range(2)],
            [pltpu.SemaphoreType.DMA for _ in range(3)],
            pltpu.SemaphoreType.DMA,
        ],
    )
    out = run(x.astype(jnp.int32), wte, wpe)
    return out.reshape(BATCH, SEQ, N_EMBD)


# speedup vs baseline: 1.0102x; 1.0025x over previous
import jax
import jax.numpy as jnp
from jax import lax
from jax.experimental import pallas as pl
from jax.experimental.pallas import tpu as pltpu
from jax.experimental.pallas import tpu_sc as plsc

VOCAB = 100000
N_EMBD = 1024
BLOCK = 2048
BATCH = 4
SEQ = 2048

NC = 2
NS = 16
NW = NC * NS
LANES = 16
S_PER_W = SEQ // NW
P = 8
NSTEP = S_PER_W // P
ROWS = BATCH * P
VPR = N_EMBD // LANES


def _emb_body(xp_hbm, oidx_hbm, wte_hbm, wpe_hbm, out_hbm,
              ridx_v, oidx_v, tok, pos, gsem, psem, ssem, isem):
    wid = lax.axis_index("s") * NC + lax.axis_index("c")
    sbase = wid * S_PER_W

    pdesc = {}
    for t in range(2):
        pdesc[t] = pltpu.async_copy(
            wpe_hbm.at[pl.ds(sbase + t * P, P)], pos[t], psem[t])

    d1 = pltpu.async_copy(xp_hbm.at[wid], ridx_v, isem)
    d2 = pltpu.async_copy(oidx_hbm.at[wid], oidx_v, isem)
    d1.wait()
    d2.wait()

    def issue_gather(t):
        return pltpu.async_copy(
            wte_hbm.at[ridx_v.at[pl.ds(t * ROWS, ROWS)]],
            tok[t % 3], gsem[t % 3])

    gdesc = {0: issue_gather(0), 1: issue_gather(1)}
    sdesc = {}
    for t in range(NSTEP):
        gdesc[t].wait()
        pdesc[t].wait()

        tb, pb = tok[t % 3], pos[t % 2]

        @plsc.parallel_loop(0, P)
        def _row(r):
            @plsc.parallel_loop(0, VPR, unroll=8)
            def _vec(k):
                sl = pl.ds(k * LANES, LANES)
                pv = pb[r, sl]
                for b in range(BATCH):
                    tb[b * P + r, sl] = tb[b * P + r, sl] + pv

        sdesc[t] = pltpu.async_copy(
            tb, out_hbm.at[oidx_v.at[t]], ssem[t % 3])

        if t + 2 < NSTEP:
            pdesc[t + 2] = pltpu.async_copy(
                wpe_hbm.at[pl.ds(sbase + (t + 2) * P, P)],
                pos[t % 2], psem[t % 2])

        if t == 0:
            gdesc[2] = issue_gather(2)
        elif t + 2 < NSTEP:
            sdesc[t - 1].wait()
            gdesc[t + 2] = issue_gather(t + 2)

    for t in (NSTEP - 3, NSTEP - 2, NSTEP - 1):
        sdesc[t].wait()


@jax.jit
def kernel(x, wte, wpe):
    xp = (x.astype(jnp.int32)
          .reshape(BATCH, NW, NSTEP, P)
          .transpose(1, 2, 0, 3)
          .reshape(NW, NSTEP * ROWS))
    w_ar = jnp.arange(NW, dtype=jnp.int32)
    t_ar = jnp.arange(NSTEP, dtype=jnp.int32)
    b_ar = jnp.arange(BATCH, dtype=jnp.int32)
    r_ar = jnp.arange(P, dtype=jnp.int32)
    orow = (b_ar[None, None, :, None] * SEQ
            + w_ar[:, None, None, None] * S_PER_W
            + t_ar[None, :, None, None] * P
            + r_ar[None, None, None, :]).reshape(NW, NSTEP, ROWS)

    mesh = plsc.VectorSubcoreMesh(core_axis_name="c", subcore_axis_name="s")
    run = pl.kernel(
        _emb_body,
        out_type=jax.ShapeDtypeStruct((BATCH * SEQ, N_EMBD), jnp.float32),
        mesh=mesh,
        scratch_types=[
            pltpu.VMEM((NSTEP * ROWS,), jnp.int32),
            pltpu.VMEM((NSTEP, ROWS), jnp.int32),
            [pltpu.VMEM((ROWS, N_EMBD), jnp.float32) for _ in range(3)],
            [pltpu.VMEM((P, N_EMBD), jnp.float32) for _ in range(2)],
            [pltpu.SemaphoreType.DMA for _ in range(3)],
            [pltpu.SemaphoreType.DMA for _ in range(2)],
            [pltpu.SemaphoreType.DMA for _ in range(3)],
            pltpu.SemaphoreType.DMA,
        ],
    )
    out = run(xp, orow, wte, wpe)
    return out.reshape(BATCH, SEQ, N_EMBD)


# speedup vs baseline: 1.0247x; 1.0144x over previous
import jax
import jax.numpy as jnp
from jax import lax
from jax.experimental import pallas as pl
from jax.experimental.pallas import tpu as pltpu
from jax.experimental.pallas import tpu_sc as plsc

VOCAB = 100000
N_EMBD = 1024
BLOCK = 2048
BATCH = 4
SEQ = 2048

NC = 2
NS = 16
NW = NC * NS
LANES = 16
S_PER_W = SEQ // NW
P = 8
NSTEP = S_PER_W // P
ROWS = BATCH * P
VPR = N_EMBD // LANES


def _emb_body(xp_hbm, oidx_hbm, wte_hbm, wpe_hbm, out_hbm,
              ridx_v, oidx_v, tok, pos, gsem, psem, ssem, isem):
    wid = lax.axis_index("s") * NC + lax.axis_index("c")
    sbase = wid * S_PER_W

    pdesc = {}
    for t in range(2):
        pdesc[t] = pltpu.async_copy(
            wpe_hbm.at[pl.ds(sbase + t * P, P)], pos[t], psem[t])

    d1 = pltpu.async_copy(xp_hbm.at[wid], ridx_v, isem)
    d2 = pltpu.async_copy(oidx_hbm.at[wid], oidx_v, isem)
    d1.wait()
    d2.wait()

    def issue_gather(t):
        return pltpu.async_copy(
            wte_hbm.at[ridx_v.at[pl.ds(t * ROWS, ROWS)]],
            tok[t % 3], gsem[t % 3])

    gdesc = {0: issue_gather(0), 1: issue_gather(1)}
    sdesc = {}
    for t in range(NSTEP):
        gdesc[t].wait()
        pdesc[t].wait()

        tb, pb = tok[t % 3], pos[t % 2]

        @plsc.parallel_loop(0, P)
        def _row(r):
            @plsc.parallel_loop(0, VPR, unroll=4)
            def _vec(k):
                sl = pl.ds(k * LANES, LANES)
                pv = pb[r, sl]
                for b in range(BATCH):
                    tb[b * P + r, sl] = tb[b * P + r, sl] + pv

        sdesc[t] = pltpu.async_copy(
            tb, out_hbm.at[oidx_v.at[t]], ssem[t % 3])

        if t + 2 < NSTEP:
            pdesc[t + 2] = pltpu.async_copy(
                wpe_hbm.at[pl.ds(sbase + (t + 2) * P, P)],
                pos[t % 2], psem[t % 2])

        if t == 0:
            gdesc[2] = issue_gather(2)
        elif t + 2 < NSTEP:
            sdesc[t - 1].wait()
            gdesc[t + 2] = issue_gather(t + 2)

    for t in (NSTEP - 3, NSTEP - 2, NSTEP - 1):
        sdesc[t].wait()


@jax.jit
def kernel(x, wte, wpe):
    xp = (x.astype(jnp.int32)
          .reshape(BATCH, NW, NSTEP, P)
          .transpose(1, 2, 0, 3)
          .reshape(NW, NSTEP * ROWS))
    w_ar = jnp.arange(NW, dtype=jnp.int32)
    t_ar = jnp.arange(NSTEP, dtype=jnp.int32)
    b_ar = jnp.arange(BATCH, dtype=jnp.int32)
    r_ar = jnp.arange(P, dtype=jnp.int32)
    orow = (b_ar[None, None, :, None] * SEQ
            + w_ar[:, None, None, None] * S_PER_W
            + t_ar[None, :, None, None] * P
            + r_ar[None, None, None, :]).reshape(NW, NSTEP, ROWS)

    mesh = plsc.VectorSubcoreMesh(core_axis_name="c", subcore_axis_name="s")
    run = pl.kernel(
        _emb_body,
        out_type=jax.ShapeDtypeStruct((BATCH * SEQ, N_EMBD), jnp.float32),
        mesh=mesh,
        scratch_types=[
            pltpu.VMEM((NSTEP * ROWS,), jnp.int32),
            pltpu.VMEM((NSTEP, ROWS), jnp.int32),
            [pltpu.VMEM((ROWS, N_EMBD), jnp.float32) for _ in range(3)],
            [pltpu.VMEM((P, N_EMBD), jnp.float32) for _ in range(2)],
            [pltpu.SemaphoreType.DMA for _ in range(3)],
            [pltpu.SemaphoreType.DMA for _ in range(2)],
            [pltpu.SemaphoreType.DMA for _ in range(3)],
            pltpu.SemaphoreType.DMA,
        ],
    )
    out = run(xp, orow, wte, wpe)
    return out.reshape(BATCH, SEQ, N_EMBD)
